# baseline (speedup 1.0000x reference)
import functools
import math

import jax
import jax.numpy as jnp
from jax import lax
from jax.experimental import pallas as pl
from jax.experimental.pallas import tpu as pltpu
from jax.experimental.pallas import tpu_sc as plsc

_NC = 2
_NS = 16
_NW = _NC * _NS
_L = 16
_CS = 8


def _sc_body(n_chunks, b, s, d, scale,
             x_hbm, lut_hbm, pe_hbm, out_hbm,
             idx_v, rows0, rows1, pe0, pe1,
             semg0, semg1, semp0, semp1, semw0, semw1):
    cid = lax.axis_index("c")
    sid = lax.axis_index("s")
    wid = sid * _NC + cid
    s_per_w = n_chunks * _CS
    sbase = wid * s_per_w

    rows = (rows0, rows1)
    pes = (pe0, pe1)
    semg = (semg0, semg1)
    semp = (semp0, semp1)
    semw = (semw0, semw1)

    for bi in range(b):
        pltpu.sync_copy(x_hbm.at[bi, pl.ds(sbase, s_per_w)], idx_v.at[bi])

    def issue(buf, ci):
        for bi in range(b):
            pltpu.async_copy(
                lut_hbm.at[idx_v.at[bi, pl.ds(ci * _CS, _CS)]],
                rows[buf].at[pl.ds(bi * _CS, _CS)], semg[buf])
        pltpu.async_copy(pe_hbm.at[pl.ds(sbase + ci * _CS, _CS)],
                         pes[buf], semp[buf])

    def wait_in(buf):
        pltpu.make_async_copy(
            pe_hbm.at[pl.ds(0, b * _CS)], rows[buf], semg[buf]).wait()
        pltpu.make_async_copy(pe_hbm.at[pl.ds(sbase, _CS)],
                              pes[buf], semp[buf]).wait()

    def compute(buf):
        rv = rows[buf]
        pv = pes[buf]

        def do_row(r, _):
            for dv in range(d // _L):
                sl = pl.ds(dv * _L, _L)
                p = pv[r, sl]
                for bi in range(b):
                    rv[bi * _CS + r, sl] = rv[bi * _CS + r, sl] * scale + p
            return ()

        lax.fori_loop(0, _CS, do_row, ())

    def wb_start(buf, ci):
        for bi in range(b):
            pltpu.async_copy(
                rows[buf].at[pl.ds(bi * _CS, _CS)],
                out_hbm.at[pl.ds(bi * s + sbase + ci * _CS, _CS)],
                semw[buf])

    def wb_wait(buf):
        for bi in range(b):
            pltpu.make_async_copy(
                rows[buf].at[pl.ds(bi * _CS, _CS)],
                out_hbm.at[pl.ds(bi * s + sbase, _CS)],
                semw[buf]).wait()

    issue(0, 0)
    issue(1, 1)

    n_pairs = n_chunks // 2

    def pair(t, _):
        c0 = 2 * t
        wait_in(0)
        compute(0)
        wb_start(0, c0)
        wait_in(1)
        compute(1)
        wb_start(1, c0 + 1)

        @pl.when(t < n_pairs - 1)
        def _():
            wb_wait(0)
            issue(0, c0 + 2)
            wb_wait(1)
            issue(1, c0 + 3)

        return ()

    lax.fori_loop(0, n_pairs, pair, ())
    wb_wait(0)
    wb_wait(1)


def kernel(x, lut, pe):
    b, s = x.shape
    v, d = lut.shape
    n = b * s
    s_per_w = s // _NW
    n_chunks = s_per_w // _CS
    scale = math.sqrt(d)

    per = pe[0]

    mesh = plsc.VectorSubcoreMesh(core_axis_name="c", subcore_axis_name="s")
    sc_call = functools.partial(
        pl.kernel,
        mesh=mesh,
        out_type=jax.ShapeDtypeStruct((n, d), jnp.float32),
        scratch_types=[
            pltpu.VMEM((b, s_per_w), jnp.int32),
            pltpu.VMEM((b * _CS, d), jnp.float32),
            pltpu.VMEM((b * _CS, d), jnp.float32),
            pltpu.VMEM((_CS, d), jnp.float32),
            pltpu.VMEM((_CS, d), jnp.float32),
            pltpu.SemaphoreType.DMA,
            pltpu.SemaphoreType.DMA,
            pltpu.SemaphoreType.DMA,
            pltpu.SemaphoreType.DMA,
            pltpu.SemaphoreType.DMA,
            pltpu.SemaphoreType.DMA,
        ],
    )(functools.partial(_sc_body, n_chunks, b, s, d, scale))

    out = sc_call(x, lut, per)
    return out.reshape(b, s, d)


# speedup vs baseline: 1.0835x; 1.0835x over previous
import functools
import math

import jax
import jax.numpy as jnp
from jax import lax
from jax.experimental import pallas as pl
from jax.experimental.pallas import tpu as pltpu
from jax.experimental.pallas import tpu_sc as plsc

_NC = 2
_NS = 16
_NW = _NC * _NS
_L = 16
_CS = 8


def _sc_body(n_chunks, b, s, d, scale,
             x_hbm, lut_hbm, pe_hbm, out_hbm,
             idx_v, rows0, rows1, pe0, pe1,
             semg0, semg1, semp0, semp1, semw0, semw1):
    cid = lax.axis_index("c")
    sid = lax.axis_index("s")
    wid = sid * _NC + cid
    s_per_w = n_chunks * _CS
    sbase = wid * s_per_w

    rows = (rows0, rows1)
    pes = (pe0, pe1)
    semg = (semg0, semg1)
    semp = (semp0, semp1)
    semw = (semw0, semw1)

    pltpu.sync_copy(x_hbm.at[wid], idx_v)

    def issue(buf, ci):
        pltpu.async_copy(lut_hbm.at[idx_v.at[ci]], rows[buf], semg[buf])
        pltpu.async_copy(pe_hbm.at[pl.ds(sbase + ci * _CS, _CS)],
                         pes[buf], semp[buf])

    def wait_in(buf):
        pltpu.make_async_copy(lut_hbm.at[idx_v.at[0]], rows[buf],
                              semg[buf]).wait()
        pltpu.make_async_copy(pe_hbm.at[pl.ds(sbase, _CS)],
                              pes[buf], semp[buf]).wait()

    def compute(buf):
        rv = rows[buf]
        pv = pes[buf]

        def do_row(r, _):
            for dv in range(d // _L):
                sl = pl.ds(dv * _L, _L)
                p = pv[r, sl]
                for bi in range(b):
                    rv[bi * _CS + r, sl] = rv[bi * _CS + r, sl] * scale + p
            return ()

        lax.fori_loop(0, _CS, do_row, ())

    def wb_start(buf, ci):
        for bi in range(b):
            pltpu.async_copy(
                rows[buf].at[pl.ds(bi * _CS, _CS)],
                out_hbm.at[pl.ds(bi * s + sbase + ci * _CS, _CS)],
                semw[buf])

    def wb_wait(buf):
        for bi in range(b):
            pltpu.make_async_copy(
                rows[buf].at[pl.ds(bi * _CS, _CS)],
                out_hbm.at[pl.ds(bi * s + sbase, _CS)],
                semw[buf]).wait()

    issue(0, 0)
    issue(1, 1)

    n_pairs = n_chunks // 2

    def pair(t, _):
        c0 = 2 * t
        wait_in(0)
        compute(0)
        wb_start(0, c0)

        @pl.when(t < n_pairs - 1)
        def _():
            wb_wait(0)
            issue(0, c0 + 2)

        wait_in(1)
        compute(1)
        wb_start(1, c0 + 1)

        @pl.when(t < n_pairs - 1)
        def _():
            wb_wait(1)
            issue(1, c0 + 3)

        return ()

    lax.fori_loop(0, n_pairs, pair, ())
    wb_wait(0)
    wb_wait(1)


def kernel(x, lut, pe):
    b, s = x.shape
    v, d = lut.shape
    n = b * s
    s_per_w = s // _NW
    n_chunks = s_per_w // _CS
    scale = math.sqrt(d)

    xc = (x.reshape(b, _NW, n_chunks, _CS)
           .transpose(1, 2, 0, 3)
           .reshape(_NW, n_chunks, b * _CS))
    per = pe[0]

    mesh = plsc.VectorSubcoreMesh(core_axis_name="c", subcore_axis_name="s")
    sc_call = functools.partial(
        pl.kernel,
        mesh=mesh,
        out_type=jax.ShapeDtypeStruct((n, d), jnp.float32),
        scratch_types=[
            pltpu.VMEM((n_chunks, b * _CS), jnp.int32),
            pltpu.VMEM((b * _CS, d), jnp.float32),
            pltpu.VMEM((b * _CS, d), jnp.float32),
            pltpu.VMEM((_CS, d), jnp.float32),
            pltpu.VMEM((_CS, d), jnp.float32),
            pltpu.SemaphoreType.DMA,
            pltpu.SemaphoreType.DMA,
            pltpu.SemaphoreType.DMA,
            pltpu.SemaphoreType.DMA,
            pltpu.SemaphoreType.DMA,
            pltpu.SemaphoreType.DMA,
        ],
    )(functools.partial(_sc_body, n_chunks, b, s, d, scale))

    out = sc_call(xc, lut, per)
    return out.reshape(b, s, d)


# speedup vs baseline: 1.0920x; 1.0078x over previous
import functools
import math

import jax
import jax.numpy as jnp
from jax import lax
from jax.experimental import pallas as pl
from jax.experimental.pallas import tpu as pltpu
from jax.experimental.pallas import tpu_sc as plsc

_NC = 2
_NS = 16
_NW = _NC * _NS
_L = 16
_CS = 8


def _sc_body(n_chunks, b, s, d, scale,
             x_hbm, lut_hbm, pe_hbm, out_hbm,
             idx_v, rows0, rows1, pe0, pe1,
             semg0, semg1, semp0, semp1, semw0, semw1):
    cid = lax.axis_index("c")
    sid = lax.axis_index("s")
    wid = sid * _NC + cid
    s_per_w = n_chunks * _CS
    sbase = wid * s_per_w

    rows = (rows0, rows1)
    pes = (pe0, pe1)
    semg = (semg0, semg1)
    semp = (semp0, semp1)
    semw = (semw0, semw1)

    pltpu.sync_copy(x_hbm.at[wid], idx_v)

    def issue(buf, ci):
        pltpu.async_copy(lut_hbm.at[idx_v.at[ci]], rows[buf], semg[buf])
        pltpu.async_copy(pe_hbm.at[pl.ds(sbase + ci * _CS, _CS)],
                         pes[buf], semp[buf])

    def drain_refill(buf, ci):
        for bi in range(b):
            pltpu.make_async_copy(
                rows[buf].at[pl.ds(bi * _CS, _CS)],
                out_hbm.at[pl.ds(bi * s + sbase, _CS)],
                semw[buf]).wait()
            pltpu.async_copy(
                lut_hbm.at[idx_v.at[ci, pl.ds(bi * _CS, _CS)]],
                rows[buf].at[pl.ds(bi * _CS, _CS)], semg[buf])
        pltpu.async_copy(pe_hbm.at[pl.ds(sbase + ci * _CS, _CS)],
                         pes[buf], semp[buf])

    def wait_in(buf):
        pltpu.make_async_copy(lut_hbm.at[idx_v.at[0]], rows[buf],
                              semg[buf]).wait()
        pltpu.make_async_copy(pe_hbm.at[pl.ds(sbase, _CS)],
                              pes[buf], semp[buf]).wait()

    def compute(buf):
        rv = rows[buf]
        pv = pes[buf]

        def do_row(r, _):
            for dv in range(d // _L):
                sl = pl.ds(dv * _L, _L)
                p = pv[r, sl]
                for bi in range(b):
                    rv[bi * _CS + r, sl] = rv[bi * _CS + r, sl] * scale + p
            return ()

        lax.fori_loop(0, _CS, do_row, ())

    def wb_start(buf, ci):
        for bi in range(b):
            pltpu.async_copy(
                rows[buf].at[pl.ds(bi * _CS, _CS)],
                out_hbm.at[pl.ds(bi * s + sbase + ci * _CS, _CS)],
                semw[buf])

    def wb_wait(buf):
        for bi in range(b):
            pltpu.make_async_copy(
                rows[buf].at[pl.ds(bi * _CS, _CS)],
                out_hbm.at[pl.ds(bi * s + sbase, _CS)],
                semw[buf]).wait()

    issue(0, 0)
    issue(1, 1)

    n_pairs = n_chunks // 2

    def pair(t, _):
        c0 = 2 * t
        wait_in(0)
        compute(0)
        wb_start(0, c0)

        @pl.when(t < n_pairs - 1)
        def _():
            drain_refill(0, c0 + 2)

        wait_in(1)
        compute(1)
        wb_start(1, c0 + 1)

        @pl.when(t < n_pairs - 1)
        def _():
            drain_refill(1, c0 + 3)

        return ()

    lax.fori_loop(0, n_pairs, pair, ())
    wb_wait(0)
    wb_wait(1)


def kernel(x, lut, pe):
    b, s = x.shape
    v, d = lut.shape
    n = b * s
    s_per_w = s // _NW
    n_chunks = s_per_w // _CS
    scale = math.sqrt(d)

    xc = (x.reshape(b, _NW, n_chunks, _CS)
           .transpose(1, 2, 0, 3)
           .reshape(_NW, n_chunks, b * _CS))
    per = pe[0]

    mesh = plsc.VectorSubcoreMesh(core_axis_name="c", subcore_axis_name="s")
    sc_call = functools.partial(
        pl.kernel,
        mesh=mesh,
        out_type=jax.ShapeDtypeStruct((n, d), jnp.float32),
        scratch_types=[
            pltpu.VMEM((n_chunks, b * _CS), jnp.int32),
            pltpu.VMEM((b * _CS, d), jnp.float32),
            pltpu.VMEM((b * _CS, d), jnp.float32),
            pltpu.VMEM((_CS, d), jnp.float32),
            pltpu.VMEM((_CS, d), jnp.float32),
            pltpu.SemaphoreType.DMA,
            pltpu.SemaphoreType.DMA,
            pltpu.SemaphoreType.DMA,
            pltpu.SemaphoreType.DMA,
            pltpu.SemaphoreType.DMA,
            pltpu.SemaphoreType.DMA,
        ],
    )(functools.partial(_sc_body, n_chunks, b, s, d, scale))

    out = sc_call(xc, lut, per)
    return out.reshape(b, s, d)
